# baseline (speedup 1.0000x reference)
import jax
import jax.numpy as jnp
from jax.experimental import pallas as pl

_B = 32
_N = 20000
_C = 25


def _decode_block(p_ref, pb_ref, o_ref):
    c = pl.program_id(0)
    x = p_ref[0]
    w = pb_ref[2:3, :]
    h = pb_ref[3:4, :]

    @pl.when(c == 0)
    def _():
        o_ref[0] = (x + 1.0) * w

    @pl.when(c == 1)
    def _():
        o_ref[0] = (x + 1.0) * h

    @pl.when(c == 2)
    def _():
        o_ref[0] = jnp.exp(x) * w

    @pl.when(c == 3)
    def _():
        o_ref[0] = jnp.exp(x) * h

    @pl.when(c >= 4)
    def _():
        o_ref[0] = x


def kernel(p, priorbox):
    pt = jnp.transpose(p, (2, 0, 1))
    pbt = jnp.transpose(priorbox, (1, 0))
    out_t = pl.pallas_call(
        _decode_block,
        grid=(_C,),
        in_specs=[
            pl.BlockSpec((1, _B, _N), lambda c: (c, 0, 0)),
            pl.BlockSpec((4, _N), lambda c: (0, 0)),
        ],
        out_specs=pl.BlockSpec((1, _B, _N), lambda c: (c, 0, 0)),
        out_shape=jax.ShapeDtypeStruct((_C, _B, _N), jnp.float32),
    )(pt, pbt)
    return jnp.transpose(out_t, (1, 2, 0))


# speedup vs baseline: 1.1472x; 1.1472x over previous
import jax
import jax.numpy as jnp
from jax.experimental import pallas as pl

_B = 32
_N = 20000
_C = 25
_PL = 4480


def _decode_block(p_ref, pb_ref, o_ref):
    w = pb_ref[2:3, :]
    h = pb_ref[3:4, :]
    o_ref[0] = (p_ref[0] + 1.0) * w
    o_ref[1] = (p_ref[1] + 1.0) * h
    o_ref[2] = jnp.exp(p_ref[2]) * w
    o_ref[3] = jnp.exp(p_ref[3]) * h
    o_ref[4:] = p_ref[4:]


def kernel(p, priorbox):
    pt = jnp.transpose(p, (2, 0, 1))
    pbt = jnp.transpose(priorbox, (1, 0))
    out_t = pl.pallas_call(
        _decode_block,
        grid=(pl.cdiv(_N, _PL),),
        in_specs=[
            pl.BlockSpec((_C, _B, _PL), lambda i: (0, 0, i)),
            pl.BlockSpec((4, _PL), lambda i: (0, i)),
        ],
        out_specs=pl.BlockSpec((_C, _B, _PL), lambda i: (0, 0, i)),
        out_shape=jax.ShapeDtypeStruct((_C, _B, _N), jnp.float32),
    )(pt, pbt)
    return jnp.transpose(out_t, (1, 2, 0))
